# baseline (speedup 1.0000x reference)
import functools

import jax
import jax.numpy as jnp
from jax import lax
from jax.experimental import pallas as pl
from jax.experimental.pallas import tpu as pltpu
from jax.experimental.pallas import tpu_sc as plsc

K = 8
E = 64


def _logits_block_kernel(x_ref, w1_ref, b1_ref, w2_ref, b2_ref, out_ref):
    h = jnp.dot(x_ref[...], w1_ref[...], preferred_element_type=jnp.float32)
    h = jnp.maximum(h + b1_ref[...], 0.0)
    logits = jnp.dot(h, w2_ref[...], preferred_element_type=jnp.float32)
    out_ref[...] = logits + b2_ref[...]


def _tc_logits(x, W1, b1, W2, b2, block_rows=1024):
    n, d = x.shape
    h_dim = W1.shape[1]
    e = W2.shape[1]
    return pl.pallas_call(
        _logits_block_kernel,
        grid=(n // block_rows,),
        in_specs=[
            pl.BlockSpec((block_rows, d), lambda i: (i, 0)),
            pl.BlockSpec((d, h_dim), lambda i: (0, 0)),
            pl.BlockSpec((1, h_dim), lambda i: (0, 0)),
            pl.BlockSpec((h_dim, e), lambda i: (0, 0)),
            pl.BlockSpec((1, e), lambda i: (0, 0)),
        ],
        out_specs=pl.BlockSpec((block_rows, e), lambda i: (i, 0)),
        out_shape=jax.ShapeDtypeStruct((n, e), jnp.float32),
    )(x, W1, b1.reshape(1, -1), W2, b2.reshape(1, -1))


def _make_sc_routing(n):
    nw = 32
    rows_per_w = n // nw
    ch = 256
    n_chunks = rows_per_w // ch
    mesh = plsc.VectorSubcoreMesh(core_axis_name="c", subcore_axis_name="s")

    @functools.partial(
        pl.kernel,
        out_type=[
            jax.ShapeDtypeStruct((n * E,), jnp.float32),
            jax.ShapeDtypeStruct((n * K,), jnp.int32),
        ],
        mesh=mesh,
        scratch_types=[
            pltpu.VMEM((ch * E,), jnp.float32),
            pltpu.VMEM((ch * E,), jnp.float32),
            pltpu.VMEM((ch * K + 16,), jnp.int32),
        ],
        compiler_params=pltpu.CompilerParams(needs_layout_passes=False),
    )
    def sc_routing(lg_hbm, gate_hbm, idx_hbm, lg_v, gate_v, idx_v):
        wid = lax.axis_index("s") * 2 + lax.axis_index("c")
        base = wid * rows_per_w
        lane = lax.iota(jnp.int32, 16)
        mask8 = lane < 8
        zero16 = jnp.zeros((16,), jnp.float32)

        def merge(ka, va, kb, vb):
            rk = lax.rev(kb, (0,))
            rv = lax.rev(vb, (0,))
            mk = jnp.where(mask8, ka, rk)
            mv = jnp.where(mask8, va, rv)
            return plsc.sort_key_val(mk, mv, descending=True)

        def row_body(r, _):
            o = r * E
            ks = []
            vs = []
            for j in range(4):
                kj = lg_v[pl.ds(o + 16 * j, 16)]
                sk, sv = plsc.sort_key_val(kj, lane + 16 * j, descending=True)
                ks.append(sk)
                vs.append(sv)
            k01, v01 = merge(ks[0], vs[0], ks[1], vs[1])
            k23, v23 = merge(ks[2], vs[2], ks[3], vs[3])
            fk, fv = merge(k01, v01, k23, v23)

            top1 = jnp.max(fk)
            p = jnp.where(mask8, jnp.exp(fk - top1), 0.0)
            w = p / jnp.sum(p)

            for j in range(4):
                gate_v[pl.ds(o + 16 * j, 16)] = zero16
            plsc.store_scatter(gate_v, [fv + o], w, mask=mask8)
            idx_v[pl.ds(r * K, 16)] = fv
            return 0

        def chunk_body(ci, _):
            start = base + ci * ch
            pltpu.sync_copy(lg_hbm.at[pl.ds(start * E, ch * E)], lg_v)
            lax.fori_loop(0, ch, row_body, 0)
            pltpu.sync_copy(gate_v, gate_hbm.at[pl.ds(start * E, ch * E)])
            pltpu.sync_copy(idx_v.at[pl.ds(0, ch * K)],
                            idx_hbm.at[pl.ds(start * K, ch * K)])
            return 0

        lax.fori_loop(0, n_chunks, chunk_body, 0)

    return sc_routing


@jax.jit
def _gating(x, W1, b1, W2, b2):
    n = x.shape[0]
    logits = _tc_logits(x, W1, b1, W2, b2)
    gate_flat, idx_flat = _make_sc_routing(n)(logits.reshape(-1))
    return gate_flat.reshape(n, E), idx_flat.reshape(n, K)


def kernel(x, W1, b1, W2, b2):
    return _gating(x, W1, b1, W2, b2)


# speedup vs baseline: 1.5727x; 1.5727x over previous
import functools

import jax
import jax.numpy as jnp
from jax.experimental import pallas as pl

K = 8


def _gating_block_kernel(x_ref, w1_ref, b1_ref, w2_ref, b2_ref,
                         gate_ref, idx_ref):
    h = jnp.dot(x_ref[...], w1_ref[...], preferred_element_type=jnp.float32)
    h = jnp.maximum(h + b1_ref[...], 0.0)
    logits = jnp.dot(h, w2_ref[...], preferred_element_type=jnp.float32)
    logits = logits + b2_ref[...]

    lt = logits.T
    e, r = lt.shape
    iota = jax.lax.broadcasted_iota(jnp.int32, (e, r), 0)
    neg_inf = jnp.float32(-jnp.inf)

    work = lt
    selected = jnp.zeros((e, r), dtype=jnp.bool_)
    idx_rows = []
    top1 = None
    for k in range(K):
        m = jnp.max(work, axis=0, keepdims=True)
        if k == 0:
            top1 = m
        is_max = work == m
        idx = jnp.min(jnp.where(is_max, iota, e), axis=0, keepdims=True)
        idx_rows.append(idx)
        one_hot = iota == idx
        selected = jnp.logical_or(selected, one_hot)
        work = jnp.where(one_hot, neg_inf, work)

    p = jnp.where(selected, jnp.exp(lt - top1), 0.0)
    z = jnp.sum(p, axis=0, keepdims=True)
    gate_ref[...] = (p / z).T
    idx_ref[...] = jnp.concatenate(idx_rows, axis=0).T


@functools.partial(jax.jit, static_argnames=("block_rows",))
def _gating(x, W1, b1, W2, b2, block_rows=1024):
    n, d = x.shape
    h_dim = W1.shape[1]
    e = W2.shape[1]
    grid = (n // block_rows,)
    gate, idx = pl.pallas_call(
        _gating_block_kernel,
        grid=grid,
        in_specs=[
            pl.BlockSpec((block_rows, d), lambda i: (i, 0)),
            pl.BlockSpec((d, h_dim), lambda i: (0, 0)),
            pl.BlockSpec((1, h_dim), lambda i: (0, 0)),
            pl.BlockSpec((h_dim, e), lambda i: (0, 0)),
            pl.BlockSpec((1, e), lambda i: (0, 0)),
        ],
        out_specs=[
            pl.BlockSpec((block_rows, e), lambda i: (i, 0)),
            pl.BlockSpec((block_rows, K), lambda i: (i, 0)),
        ],
        out_shape=[
            jax.ShapeDtypeStruct((n, e), jnp.float32),
            jax.ShapeDtypeStruct((n, K), jnp.int32),
        ],
    )(x, W1, b1.reshape(1, -1), W2, b2.reshape(1, -1))
    return gate, idx


def kernel(x, W1, b1, W2, b2):
    return _gating(x, W1, b1, W2, b2)
